# initial kernel scaffold (unmeasured)
import jax
import jax.numpy as jnp
from jax import lax
from jax.experimental import pallas as pl
from jax.experimental.pallas import tpu as pltpu


def kernel(
    x,
):
    def body(*refs):
        pass

    out_shape = jax.ShapeDtypeStruct(..., jnp.float32)
    return pl.pallas_call(body, out_shape=out_shape)(...)


# baseline (device time: 12795 ns/iter reference)
import jax
import jax.numpy as jnp
from jax import lax
from jax.experimental import pallas as pl
from jax.experimental.pallas import tpu as pltpu

N_DEV = 16


def kernel(x):
    m_per, n = x.shape
    inv_m = 1.0 / float(N_DEV * m_per)

    def body(x_ref, out_ref, partial_ref, comm_ref, send_sems, recv_sems):
        my_pos = lax.axis_index("i")

        partial_ref[...] = jnp.sum(
            x_ref[...].astype(jnp.float32), axis=0, keepdims=True
        )

        barrier_sem = pltpu.get_barrier_semaphore()
        for d in range(1, N_DEV):
            pl.semaphore_signal(
                barrier_sem,
                inc=1,
                device_id=((my_pos + d) % N_DEV,),
                device_id_type=pl.DeviceIdType.MESH,
            )
        pl.semaphore_wait(barrier_sem, N_DEV - 1)

        rdmas = []
        for d in range(1, N_DEV):
            rdma = pltpu.make_async_remote_copy(
                src_ref=partial_ref,
                dst_ref=comm_ref.at[d - 1],
                send_sem=send_sems.at[d - 1],
                recv_sem=recv_sems.at[d - 1],
                device_id=((my_pos + d) % N_DEV,),
                device_id_type=pl.DeviceIdType.MESH,
            )
            rdma.start()
            rdmas.append(rdma)

        acc = partial_ref[...]
        for d in range(1, N_DEV):
            rdmas[d - 1].wait_recv()
            acc = acc + comm_ref[d - 1]
        out_ref[...] = acc * inv_m

        for d in range(1, N_DEV):
            rdmas[d - 1].wait_send()

    return pl.pallas_call(
        body,
        out_shape=jax.ShapeDtypeStruct((1, n), jnp.float32),
        in_specs=[pl.BlockSpec(memory_space=pltpu.VMEM)],
        out_specs=pl.BlockSpec(memory_space=pltpu.VMEM),
        scratch_shapes=[
            pltpu.VMEM((1, n), jnp.float32),
            pltpu.VMEM((N_DEV - 1, 1, n), jnp.float32),
            pltpu.SemaphoreType.DMA((N_DEV - 1,)),
            pltpu.SemaphoreType.DMA((N_DEV - 1,)),
        ],
        compiler_params=pltpu.CompilerParams(collective_id=0),
    )(x)


# device time: 12481 ns/iter; 1.0252x vs baseline; 1.0252x over previous
import jax
import jax.numpy as jnp
from jax import lax
from jax.experimental import pallas as pl
from jax.experimental.pallas import tpu as pltpu

N_DEV = 16
BLOCK_M = 256


def kernel(x):
    m_per, n = x.shape
    n_blk = m_per // BLOCK_M
    inv_m = 1.0 / float(N_DEV * m_per)

    def body(x_ref, out_ref, partial_ref, comm_ref, send_sems, recv_sems):
        step = pl.program_id(0)
        my_pos = lax.axis_index("i")
        barrier_sem = pltpu.get_barrier_semaphore()

        @pl.when(step == 0)
        def _():
            for d in range(1, N_DEV):
                pl.semaphore_signal(
                    barrier_sem,
                    inc=1,
                    device_id=((my_pos + d) % N_DEV,),
                    device_id_type=pl.DeviceIdType.MESH,
                )

        chunk = jnp.sum(x_ref[...].astype(jnp.float32), axis=0, keepdims=True)

        @pl.when(step == 0)
        def _():
            partial_ref[...] = chunk

        @pl.when(step > 0)
        def _():
            partial_ref[...] += chunk

        @pl.when(step == n_blk - 1)
        def _():
            pl.semaphore_wait(barrier_sem, N_DEV - 1)

            rdmas = []
            for d in range(1, N_DEV):
                rdma = pltpu.make_async_remote_copy(
                    src_ref=partial_ref,
                    dst_ref=comm_ref.at[d - 1],
                    send_sem=send_sems.at[d - 1],
                    recv_sem=recv_sems.at[d - 1],
                    device_id=((my_pos + d) % N_DEV,),
                    device_id_type=pl.DeviceIdType.MESH,
                )
                rdma.start()
                rdmas.append(rdma)

            acc = partial_ref[...]
            for d in range(1, N_DEV):
                rdmas[d - 1].wait_recv()
                acc = acc + comm_ref[d - 1]
            out_ref[...] = acc * inv_m

            for d in range(1, N_DEV):
                rdmas[d - 1].wait_send()

    return pl.pallas_call(
        body,
        grid=(n_blk,),
        out_shape=jax.ShapeDtypeStruct((1, n), jnp.float32),
        in_specs=[pl.BlockSpec((BLOCK_M, n), lambda i: (i, 0))],
        out_specs=pl.BlockSpec((1, n), lambda i: (0, 0)),
        scratch_shapes=[
            pltpu.VMEM((1, n), jnp.float32),
            pltpu.VMEM((N_DEV - 1, 1, n), jnp.float32),
            pltpu.SemaphoreType.DMA((N_DEV - 1,)),
            pltpu.SemaphoreType.DMA((N_DEV - 1,)),
        ],
        compiler_params=pltpu.CompilerParams(
            collective_id=0,
            dimension_semantics=("arbitrary",),
        ),
    )(x)
